# baseline (speedup 1.0000x reference)
import functools

import jax
import jax.numpy as jnp
from jax import lax
from jax.experimental import pallas as pl
from jax.experimental.pallas import tpu as pltpu
from jax.experimental.pallas import tpu_sc as plsc

N_NODES = 10000
N_EDGES = 320000
D = 128

NC = 2
NS = 16
NW = NC * NS

N_PAD = 10240
STRIPE = N_PAD // NS

CHUNK = 128
K_PER_W = 80
N_CHUNKS = NW * K_PER_W
E_PAD = N_CHUNKS * CHUNK


def _mm_body(x_ref, w_ref, b_ref, o_ref):
    o_ref[...] = (
        lax.dot_general(
            x_ref[...], w_ref[...], (((1,), (1,)), ((), ())),
            preferred_element_type=jnp.float32,
        )
        + b_ref[...]
    )


def _linear(x, W, b):
    m_blk = 1000
    return pl.pallas_call(
        _mm_body,
        grid=(N_NODES // m_blk,),
        in_specs=[
            pl.BlockSpec((m_blk, D), lambda i: (i, 0)),
            pl.BlockSpec((D, D), lambda i: (0, 0)),
            pl.BlockSpec((1, D), lambda i: (0, 0)),
        ],
        out_specs=pl.BlockSpec((m_blk, D), lambda i: (i, 0)),
        out_shape=jax.ShapeDtypeStruct((N_NODES, D), jnp.float32),
    )(x, W, b.reshape(1, D))


def _add_body(p0_ref, p1_ref, o_ref):
    o_ref[...] = p0_ref[...] + p1_ref[...]


def _combine(partials):
    m_blk = 1024
    out = pl.pallas_call(
        _add_body,
        grid=(N_PAD // m_blk,),
        in_specs=[
            pl.BlockSpec((m_blk, D), lambda i: (i, 0)),
            pl.BlockSpec((m_blk, D), lambda i: (i + N_PAD // m_blk, 0)),
        ],
        out_specs=pl.BlockSpec((m_blk, D), lambda i: (i, 0)),
        out_shape=jax.ShapeDtypeStruct((N_PAD, D), jnp.float32),
    )(partials, partials)
    return out[:N_NODES]


def _sc_aggregate(out, src, dst, zeros):
    mesh = plsc.VectorSubcoreMesh(core_axis_name="c", subcore_axis_name="s")

    @functools.partial(
        pl.kernel,
        mesh=mesh,
        out_type=jax.ShapeDtypeStruct((NC * N_PAD, D), jnp.float32),
        scratch_types=[
            pltpu.VMEM((CHUNK,), jnp.int32),
            pltpu.VMEM((CHUNK,), jnp.int32),
            pltpu.VMEM((CHUNK, D), jnp.float32),
            pltpu.VMEM_SHARED((N_PAD, D), jnp.float32),
            pltpu.SemaphoreType.DMA,
        ],
    )
    def k(out_hbm, src_hbm, dst_hbm, zero_hbm, o_hbm, srcv, dstv, rows, acc, sem):
        cid = lax.axis_index("c")
        sid = lax.axis_index("s")
        wid = sid * NC + cid

        pltpu.sync_copy(zero_hbm, acc.at[pl.ds(sid * STRIPE, STRIPE)])
        plsc.subcore_barrier()

        @pl.loop(wid, N_CHUNKS, step=NW)
        def _(c):
            base = c * CHUNK
            pltpu.sync_copy(src_hbm.at[pl.ds(base, CHUNK)], srcv)
            pltpu.sync_copy(dst_hbm.at[pl.ds(base, CHUNK)], dstv)
            pltpu.async_copy(out_hbm.at[srcv], rows, sem).wait()
            pltpu.sync_copy(rows, acc.at[dstv], add=True)

        plsc.subcore_barrier()
        pltpu.sync_copy(
            acc.at[pl.ds(sid * STRIPE, STRIPE)],
            o_hbm.at[pl.ds(cid * N_PAD + sid * STRIPE, STRIPE)],
        )

    return k(out, src, dst, zeros)


def kernel(x, edge_index, W, b):
    ei = edge_index.astype(jnp.int32)
    pad = E_PAD - N_EDGES
    src = jnp.concatenate([ei[0], jnp.zeros((pad,), jnp.int32)])
    pad_dst = N_NODES + jnp.arange(pad, dtype=jnp.int32) % (N_PAD - N_NODES)
    dst = jnp.concatenate([ei[1], pad_dst])
    out = _linear(x, W, b)
    zeros = jnp.zeros((STRIPE, D), jnp.float32)
    partials = _sc_aggregate(out, src, dst, zeros)
    return _combine(partials)


# speedup vs baseline: 1.9040x; 1.9040x over previous
import functools

import jax
import jax.numpy as jnp
from jax import lax
from jax.experimental import pallas as pl
from jax.experimental.pallas import tpu as pltpu
from jax.experimental.pallas import tpu_sc as plsc

N_NODES = 10000
N_EDGES = 320000
D = 128

NC = 2
NS = 16
NW = NC * NS

N_PAD = 10240
STRIPE = N_PAD // NS

CHUNK = 128
K_PER_W = 80
N_CHUNKS = NW * K_PER_W
E_PAD = N_CHUNKS * CHUNK


def _mm_body(x_ref, w_ref, b_ref, o_ref):
    o_ref[...] = (
        lax.dot_general(
            x_ref[...], w_ref[...], (((1,), (1,)), ((), ())),
            preferred_element_type=jnp.float32,
        )
        + b_ref[...]
    )


def _linear(x, W, b):
    m_blk = 1000
    return pl.pallas_call(
        _mm_body,
        grid=(N_NODES // m_blk,),
        in_specs=[
            pl.BlockSpec((m_blk, D), lambda i: (i, 0)),
            pl.BlockSpec((D, D), lambda i: (0, 0)),
            pl.BlockSpec((1, D), lambda i: (0, 0)),
        ],
        out_specs=pl.BlockSpec((m_blk, D), lambda i: (i, 0)),
        out_shape=jax.ShapeDtypeStruct((N_NODES, D), jnp.float32),
    )(x, W, b.reshape(1, D))


def _add_body(p0_ref, p1_ref, o_ref):
    o_ref[...] = p0_ref[...] + p1_ref[...]


def _combine(partials):
    m_blk = 1024
    out = pl.pallas_call(
        _add_body,
        grid=(N_PAD // m_blk,),
        in_specs=[
            pl.BlockSpec((m_blk, D), lambda i: (i, 0)),
            pl.BlockSpec((m_blk, D), lambda i: (i + N_PAD // m_blk, 0)),
        ],
        out_specs=pl.BlockSpec((m_blk, D), lambda i: (i, 0)),
        out_shape=jax.ShapeDtypeStruct((N_PAD, D), jnp.float32),
    )(partials, partials)
    return out[:N_NODES]


def _sc_aggregate(out, src, dst, zeros):
    mesh = plsc.VectorSubcoreMesh(core_axis_name="c", subcore_axis_name="s")

    @functools.partial(
        pl.kernel,
        mesh=mesh,
        out_type=jax.ShapeDtypeStruct((NC * N_PAD, D), jnp.float32),
        scratch_types=[
            pltpu.VMEM((CHUNK,), jnp.int32),
            pltpu.VMEM((CHUNK,), jnp.int32),
            pltpu.VMEM((CHUNK, D), jnp.float32),
            pltpu.VMEM_SHARED((N_PAD, D), jnp.float32),
            pltpu.SemaphoreType.DMA,
        ],
    )
    def k(out_hbm, src_hbm, dst_hbm, zero_hbm, o_hbm, srcv, dstv, rows, acc, sem):
        cid = lax.axis_index("c")
        sid = lax.axis_index("s")
        wid = sid * NC + cid

        pltpu.sync_copy(zero_hbm, acc.at[pl.ds(sid * STRIPE, STRIPE)])
        plsc.subcore_barrier()

        @pl.loop(wid, N_CHUNKS, step=NW)
        def _(c):
            base = c * CHUNK
            pltpu.sync_copy(src_hbm.at[pl.ds(base, CHUNK)], srcv)
            pltpu.sync_copy(dst_hbm.at[pl.ds(base, CHUNK)], dstv)
            pltpu.async_copy(out_hbm.at[srcv], rows, sem).wait()
            pltpu.sync_copy(rows, acc.at[dstv], add=True)

        plsc.subcore_barrier()
        pltpu.sync_copy(
            acc.at[pl.ds(sid * STRIPE, STRIPE)],
            o_hbm.at[pl.ds(cid * N_PAD + sid * STRIPE, STRIPE)],
        )

    return k(out, src, dst, zeros)


def kernel(x, edge_index, W, b):
    ei = edge_index.astype(jnp.int32)
    pad = E_PAD - N_EDGES
    pad_src = jnp.arange(pad, dtype=jnp.int32) % N_NODES
    src = jnp.concatenate([ei[0], pad_src])
    pad_dst = N_NODES + jnp.arange(pad, dtype=jnp.int32) % (N_PAD - N_NODES)
    dst = jnp.concatenate([ei[1], pad_dst])
    out = _linear(x, W, b)
    zeros = jnp.zeros((STRIPE, D), jnp.float32)
    partials = _sc_aggregate(out, src, dst, zeros)
    return _combine(partials)


# speedup vs baseline: 2.5040x; 1.3151x over previous
import functools

import jax
import jax.numpy as jnp
from jax import lax
from jax.experimental import pallas as pl
from jax.experimental.pallas import tpu as pltpu
from jax.experimental.pallas import tpu_sc as plsc

N_NODES = 10000
N_EDGES = 320000
D = 128

NC = 2
NS = 16
NW = NC * NS

N_PAD = 10240
STRIPE = N_PAD // NS

CHUNK = 128
K_PER_W = 80
N_CHUNKS = NW * K_PER_W
E_PAD = N_CHUNKS * CHUNK


def _mm_body(x_ref, w_ref, b_ref, o_ref):
    o_ref[...] = (
        lax.dot_general(
            x_ref[...], w_ref[...], (((1,), (1,)), ((), ())),
            preferred_element_type=jnp.float32,
        )
        + b_ref[...]
    )


def _linear(x, W, b):
    m_blk = 1000
    return pl.pallas_call(
        _mm_body,
        grid=(N_NODES // m_blk,),
        in_specs=[
            pl.BlockSpec((m_blk, D), lambda i: (i, 0)),
            pl.BlockSpec((D, D), lambda i: (0, 0)),
            pl.BlockSpec((1, D), lambda i: (0, 0)),
        ],
        out_specs=pl.BlockSpec((m_blk, D), lambda i: (i, 0)),
        out_shape=jax.ShapeDtypeStruct((N_NODES, D), jnp.float32),
    )(x, W, b.reshape(1, D))


def _add_body(p0_ref, p1_ref, o_ref):
    o_ref[...] = p0_ref[...] + p1_ref[...]


def _combine(partials):
    m_blk = 1024
    out = pl.pallas_call(
        _add_body,
        grid=(N_PAD // m_blk,),
        in_specs=[
            pl.BlockSpec((m_blk, D), lambda i: (i, 0)),
            pl.BlockSpec((m_blk, D), lambda i: (i + N_PAD // m_blk, 0)),
        ],
        out_specs=pl.BlockSpec((m_blk, D), lambda i: (i, 0)),
        out_shape=jax.ShapeDtypeStruct((N_PAD, D), jnp.float32),
    )(partials, partials)
    return out[:N_NODES]


def _sc_aggregate(out, src, dst, zeros):
    mesh = plsc.VectorSubcoreMesh(core_axis_name="c", subcore_axis_name="s")

    @functools.partial(
        pl.kernel,
        mesh=mesh,
        out_type=jax.ShapeDtypeStruct((NC * N_PAD, D), jnp.float32),
        scratch_types=[
            pltpu.VMEM((CHUNK,), jnp.int32),
            pltpu.VMEM((CHUNK,), jnp.int32),
            pltpu.VMEM((CHUNK,), jnp.int32),
            pltpu.VMEM((CHUNK,), jnp.int32),
            pltpu.VMEM((CHUNK, D), jnp.float32),
            pltpu.VMEM((CHUNK, D), jnp.float32),
            pltpu.VMEM_SHARED((N_PAD, D), jnp.float32),
            pltpu.SemaphoreType.DMA,
            pltpu.SemaphoreType.DMA,
        ],
    )
    def k(out_hbm, src_hbm, dst_hbm, zero_hbm, o_hbm,
          src_a, dst_a, src_b, dst_b, rows_a, rows_b, acc, sem_a, sem_b):
        cid = lax.axis_index("c")
        sid = lax.axis_index("s")
        wid = sid * NC + cid

        pltpu.sync_copy(zero_hbm, acc.at[pl.ds(sid * STRIPE, STRIPE)])
        plsc.subcore_barrier()

        def load_gather(c, srci, dsti, rows, sem):
            base = c * CHUNK
            pltpu.sync_copy(src_hbm.at[pl.ds(base, CHUNK)], srci)
            pltpu.sync_copy(dst_hbm.at[pl.ds(base, CHUNK)], dsti)
            return pltpu.async_copy(out_hbm.at[srci], rows, sem)

        @pl.loop(wid, N_CHUNKS, step=2 * NW)
        def _(c):
            d_a = load_gather(c, src_a, dst_a, rows_a, sem_a)
            d_b = load_gather(c + NW, src_b, dst_b, rows_b, sem_b)
            d_a.wait()
            pltpu.sync_copy(rows_a, acc.at[dst_a], add=True)
            d_b.wait()
            pltpu.sync_copy(rows_b, acc.at[dst_b], add=True)

        plsc.subcore_barrier()
        pltpu.sync_copy(
            acc.at[pl.ds(sid * STRIPE, STRIPE)],
            o_hbm.at[pl.ds(cid * N_PAD + sid * STRIPE, STRIPE)],
        )

    return k(out, src, dst, zeros)


def kernel(x, edge_index, W, b):
    ei = edge_index.astype(jnp.int32)
    pad = E_PAD - N_EDGES
    pad_src = jnp.arange(pad, dtype=jnp.int32) % N_NODES
    src = jnp.concatenate([ei[0], pad_src])
    pad_dst = N_NODES + jnp.arange(pad, dtype=jnp.int32) % (N_PAD - N_NODES)
    dst = jnp.concatenate([ei[1], pad_dst])
    out = _linear(x, W, b)
    zeros = jnp.zeros((STRIPE, D), jnp.float32)
    partials = _sc_aggregate(out, src, dst, zeros)
    return _combine(partials)


# speedup vs baseline: 3.2186x; 1.2854x over previous
import functools

import jax
import jax.numpy as jnp
from jax import lax
from jax.experimental import pallas as pl
from jax.experimental.pallas import tpu as pltpu
from jax.experimental.pallas import tpu_sc as plsc

N_NODES = 10000
N_EDGES = 320000
D = 128

NC = 2
NS = 16
NW = NC * NS

N_PAD = 10240
STRIPE = N_PAD // NS

CHUNK = 120
K_PER_W = 84
N_CHUNKS = NW * K_PER_W
E_PAD = N_CHUNKS * CHUNK


def _mm_body(x_ref, w_ref, b_ref, o_ref):
    o_ref[...] = (
        lax.dot_general(
            x_ref[...], w_ref[...], (((1,), (1,)), ((), ())),
            preferred_element_type=jnp.float32,
        )
        + b_ref[...]
    )


def _linear(x, W, b):
    m_blk = 1000
    return pl.pallas_call(
        _mm_body,
        grid=(N_NODES // m_blk,),
        in_specs=[
            pl.BlockSpec((m_blk, D), lambda i: (i, 0)),
            pl.BlockSpec((D, D), lambda i: (0, 0)),
            pl.BlockSpec((1, D), lambda i: (0, 0)),
        ],
        out_specs=pl.BlockSpec((m_blk, D), lambda i: (i, 0)),
        out_shape=jax.ShapeDtypeStruct((N_NODES, D), jnp.float32),
    )(x, W, b.reshape(1, D))


def _add_body(p0_ref, p1_ref, o_ref):
    o_ref[...] = p0_ref[...] + p1_ref[...]


def _combine(partials):
    m_blk = 1024
    out = pl.pallas_call(
        _add_body,
        grid=(N_PAD // m_blk,),
        in_specs=[
            pl.BlockSpec((m_blk, D), lambda i: (i, 0)),
            pl.BlockSpec((m_blk, D), lambda i: (i + N_PAD // m_blk, 0)),
        ],
        out_specs=pl.BlockSpec((m_blk, D), lambda i: (i, 0)),
        out_shape=jax.ShapeDtypeStruct((N_PAD, D), jnp.float32),
    )(partials, partials)
    return out[:N_NODES]


def _sc_aggregate(out, eidx, zeros):
    mesh = plsc.VectorSubcoreMesh(core_axis_name="c", subcore_axis_name="s")

    @functools.partial(
        pl.kernel,
        mesh=mesh,
        out_type=jax.ShapeDtypeStruct((NC * N_PAD, D), jnp.float32),
        scratch_types=[
            pltpu.VMEM((2, CHUNK), jnp.int32),
            pltpu.VMEM((2, CHUNK), jnp.int32),
            pltpu.VMEM((2, CHUNK), jnp.int32),
            pltpu.VMEM((CHUNK, D), jnp.float32),
            pltpu.VMEM((CHUNK, D), jnp.float32),
            pltpu.VMEM((CHUNK, D), jnp.float32),
            pltpu.VMEM_SHARED((N_PAD, D), jnp.float32),
            pltpu.SemaphoreType.DMA,
            pltpu.SemaphoreType.DMA,
            pltpu.SemaphoreType.DMA,
        ],
    )
    def k(out_hbm, eidx_hbm, zero_hbm, o_hbm,
          idx_a, idx_b, idx_c, rows_a, rows_b, rows_c, acc,
          sem_a, sem_b, sem_c):
        cid = lax.axis_index("c")
        sid = lax.axis_index("s")
        wid = sid * NC + cid

        pltpu.sync_copy(zero_hbm, acc.at[pl.ds(sid * STRIPE, STRIPE)])
        plsc.subcore_barrier()

        def load_gather(c, idxv, rows, sem):
            pltpu.sync_copy(eidx_hbm.at[c], idxv)
            pltpu.async_copy(out_hbm.at[idxv.at[0]], rows, sem)

        def drain_scatter(idxv, rows, sem):
            pltpu.make_async_copy(zero_hbm.at[pl.ds(0, CHUNK)], rows, sem).wait()
            pltpu.sync_copy(rows, acc.at[idxv.at[1]], add=True)

        load_gather(wid, idx_a, rows_a, sem_a)
        load_gather(wid + NW, idx_b, rows_b, sem_b)
        load_gather(wid + 2 * NW, idx_c, rows_c, sem_c)

        @pl.loop(wid, N_CHUNKS - 3 * NW, step=3 * NW)
        def _(c):
            drain_scatter(idx_a, rows_a, sem_a)
            load_gather(c + 3 * NW, idx_a, rows_a, sem_a)
            drain_scatter(idx_b, rows_b, sem_b)
            load_gather(c + 4 * NW, idx_b, rows_b, sem_b)
            drain_scatter(idx_c, rows_c, sem_c)
            load_gather(c + 5 * NW, idx_c, rows_c, sem_c)

        drain_scatter(idx_a, rows_a, sem_a)
        drain_scatter(idx_b, rows_b, sem_b)
        drain_scatter(idx_c, rows_c, sem_c)

        plsc.subcore_barrier()
        pltpu.sync_copy(
            acc.at[pl.ds(sid * STRIPE, STRIPE)],
            o_hbm.at[pl.ds(cid * N_PAD + sid * STRIPE, STRIPE)],
        )

    return k(out, eidx, zeros)


def kernel(x, edge_index, W, b):
    ei = edge_index.astype(jnp.int32)
    pad = E_PAD - N_EDGES
    pad_src = jnp.arange(pad, dtype=jnp.int32) % N_NODES
    src = jnp.concatenate([ei[0], pad_src]).reshape(N_CHUNKS, 1, CHUNK)
    pad_dst = N_NODES + jnp.arange(pad, dtype=jnp.int32) % (N_PAD - N_NODES)
    dst = jnp.concatenate([ei[1], pad_dst]).reshape(N_CHUNKS, 1, CHUNK)
    eidx = jnp.concatenate([src, dst], axis=1)
    out = _linear(x, W, b)
    zeros = jnp.zeros((STRIPE, D), jnp.float32)
    partials = _sc_aggregate(out, eidx, zeros)
    return _combine(partials)
